# baseline (speedup 1.0000x reference)
import functools

import jax
import jax.numpy as jnp
from jax import lax
from jax.experimental import pallas as pl
from jax.experimental.pallas import tpu as pltpu
from jax.experimental.pallas import tpu_sc as plsc

NC = 2
NS = 16
L = 16
NW = NC * NS

VOCAB = 1000000
D = 64
B = 16384
NCTX = 20
VP = VOCAB // 2

TW = 512
FULL_CHUNKS = VOCAB // TW
STEPS = (FULL_CHUNKS - 1) // NW
TAIL_V0 = FULL_CHUNKS * TW

B_PER_W = B // NW
CH_B = 8
CHUNKS = B_PER_W // CH_B
CH_ROWS = CH_B * NCTX
IDX_CHUNKS = ((0, 128), (128, 32))

_CP = pltpu.CompilerParams(needs_layout_passes=False, use_tc_tiling_on_sc=True)


def _rsqrt(s):
    i = plsc.bitcast(s, jnp.int32)
    i = 0x5F3759DF - lax.shift_right_arithmetic(i, 1)
    y = plsc.bitcast(i, jnp.float32)
    h = s * 0.5
    for _ in range(3):
        y = y * (1.5 - h * y * y)
    return y


def _pack_body(ct_t, xt_t, tailc, tailx, outc, outx, ib0, ib1, ob, sem0, sem1):
    wid = lax.axis_index("s") * NC + lax.axis_index("c")
    iota = lax.iota(jnp.int32, L)

    def fire(tab, ci, ib, sem):
        v0 = pl.multiple_of(ci * TW, TW)
        pltpu.async_copy(tab.at[:, pl.ds(v0, TW)], ib, sem)

    def drain(tab, ci, ib, sem):
        v0 = pl.multiple_of(ci * TW, TW)
        pltpu.make_async_copy(tab.at[:, pl.ds(v0, TW)], ib, sem).wait()

    def transpose_out(ci, ib, out):
        def prow(q, _):
            p16 = q * L + iota
            for h in range(2):
                for d0 in range(D):
                    dv = jnp.bitwise_and(iota + d0, D - 1)
                    v = plsc.load_gather(ib, [dv, 2 * p16 + h])
                    plsc.store_scatter(ob, [p16, dv + (D * h)], v)
            return 0
        lax.fori_loop(0, TW // 2 // L, prow, 0)
        p0 = pl.multiple_of(ci * (TW // 2), TW // 2)
        pltpu.sync_copy(ob, out.at[pl.ds(p0, TW // 2)])

    fire(ct_t, wid, ib0, sem0)

    def step(t, _):
        ci = wid + NW * t
        fire(xt_t, ci, ib1, sem1)
        drain(ct_t, ci, ib0, sem0)
        transpose_out(ci, ib0, outc)

        @pl.when(t < STEPS - 1)
        def _():
            fire(ct_t, wid + NW * (t + 1), ib0, sem0)
        drain(xt_t, ci, ib1, sem1)
        transpose_out(ci, ib1, outx)
        return 0

    lax.fori_loop(0, STEPS, step, 0)

    @pl.when(wid == 0)
    def _():
        ci = FULL_CHUNKS - 1
        for tab, out in ((ct_t, outc), (xt_t, outx)):
            pltpu.async_copy(tab.at[:, pl.ds(ci * TW, TW)], ib0, sem0).wait()
            transpose_out(ci, ib0, out)

    @pl.when(wid == 1)
    def _():
        for tail, out in ((tailc, outc), (tailx, outx)):
            pltpu.sync_copy(tail, ob.at[pl.ds(0, 32)])
            pltpu.sync_copy(ob.at[pl.ds(0, 32)],
                            out.at[pl.ds(TAIL_V0 // 2, 32)])


@jax.jit
def _sc_pack(centers_t, contexts_t, tailc, tailx):
    mesh = plsc.VectorSubcoreMesh(core_axis_name="c", subcore_axis_name="s")
    f = functools.partial(
        pl.kernel,
        out_type=(jax.ShapeDtypeStruct((VP, 128), jnp.float32),
                  jax.ShapeDtypeStruct((VP, 128), jnp.float32)),
        mesh=mesh,
        compiler_params=_CP,
        scratch_types=[
            pltpu.VMEM((D, TW), jnp.float32),
            pltpu.VMEM((D, TW), jnp.float32),
            pltpu.VMEM((TW // 2, 128), jnp.float32),
            pltpu.SemaphoreType.DMA,
            pltpu.SemaphoreType.DMA,
        ],
    )(_pack_body)
    return f(centers_t, contexts_t, tailc, tailx)


def _loss_body(pc, px, cidx, ctxidx, negidx, out,
               cidx_v, cidx2, ct, outv,
               ci0, ci1, ni0, ni1, ci2_0, ci2_1, ni2_0, ni2_1,
               cg0, cg1, cr0, cr1, nr0, nr1,
               sem_c0, sem_c1, sem_x0, sem_x1, sem_n0, sem_n1):
    wid = lax.axis_index("s") * NC + lax.axis_index("c")
    w_base = wid * B_PER_W

    iota = lax.iota(jnp.int32, L)
    lane_b = lax.shift_right_logical(iota, 1)
    lane_j = jnp.bitwise_and(iota, 1)
    rowbase = lane_b * NCTX + lane_j
    dperm = [jnp.bitwise_and(iota + d, D - 1) for d in range(D)]

    def dperm_dyn(dd):
        return jnp.bitwise_and(iota + dd, D - 1)

    pltpu.sync_copy(cidx.at[pl.ds(w_base, B_PER_W)], cidx_v)
    for k in range(B_PER_W // L):
        cidx2[pl.ds(k * L, L)] = lax.shift_right_logical(
            cidx_v[pl.ds(k * L, L)], 1)

    def fire(g, ci, ni, ci2, ni2, cg, cr, nr, sem_c, sem_x, sem_n):
        b0 = g * CH_B
        off = (w_base + b0) * NCTX
        pltpu.sync_copy(ctxidx.at[pl.ds(off, CH_ROWS)], ci)
        pltpu.sync_copy(negidx.at[pl.ds(off, CH_ROWS)], ni)
        for k in range(CH_ROWS // L):
            ci2[pl.ds(k * L, L)] = lax.shift_right_logical(
                ci[pl.ds(k * L, L)], 1)
            ni2[pl.ds(k * L, L)] = lax.shift_right_logical(
                ni[pl.ds(k * L, L)], 1)
        pltpu.async_copy(pc.at[cidx2.at[pl.ds(b0, CH_B)]], cg, sem_c)
        for s, n in IDX_CHUNKS:
            pltpu.async_copy(px.at[ci2.at[pl.ds(s, n)]],
                             cr.at[pl.ds(s, n)], sem_x)
            pltpu.async_copy(px.at[ni2.at[pl.ds(s, n)]],
                             nr.at[pl.ds(s, n)], sem_n)

    def drain(g, ci2, ni2, cg, cr, nr, sem_c, sem_x, sem_n):
        b0 = g * CH_B
        pltpu.make_async_copy(pc.at[cidx2.at[pl.ds(b0, CH_B)]],
                              cg, sem_c).wait()
        for s, n in IDX_CHUNKS:
            pltpu.make_async_copy(px.at[ci2.at[pl.ds(s, n)]],
                                  cr.at[pl.ds(s, n)], sem_x).wait()
            pltpu.make_async_copy(px.at[ni2.at[pl.ds(s, n)]],
                                  nr.at[pl.ds(s, n)], sem_n).wait()

    z = jnp.zeros((L,), jnp.float32)

    def compute(g, ci, ni, cg, cr, nr, acc):
        b0 = g * CH_B
        par_c = jnp.bitwise_and(
            plsc.load_gather(cidx_v, [b0 + lane_b]), 1)
        ccol = par_c * D

        def tbody(t, carry):
            cc0, cc1 = carry
            d0 = t * 8
            for u in range(8):
                cv = plsc.load_gather(cg, [lane_b, dperm_dyn(d0 + u) + ccol])
                ct[d0 + u] = cv
                if u % 2 == 0:
                    cc0 = cc0 + cv * cv
                else:
                    cc1 = cc1 + cv * cv
            return (cc0, cc1)
        cc0, cc1 = lax.fori_loop(0, D // 8, tbody, (z, z))
        cc = cc0 + cc1

        def jbody(jp, acc):
            rowx = rowbase + 2 * jp
            pcol = jnp.bitwise_and(plsc.load_gather(ci, [rowx]), 1) * D
            ncol = jnp.bitwise_and(plsc.load_gather(ni, [rowx]), 1) * D
            dp = [z, z]
            xxpa = [z, z]
            dn = [z, z]
            xxna = [z, z]
            for d in range(D):
                k = d & 1
                cv = ct[d]
                xp = plsc.load_gather(cr, [rowx, dperm[d] + pcol])
                xn = plsc.load_gather(nr, [rowx, dperm[d] + ncol])
                dp[k] = dp[k] + cv * xp
                xxpa[k] = xxpa[k] + xp * xp
                dn[k] = dn[k] + cv * xn
                xxna[k] = xxna[k] + xn * xn
            dotp = dp[0] + dp[1]
            xxp = xxpa[0] + xxpa[1]
            dotn = dn[0] + dn[1]
            xxn = xxna[0] + xxna[1]
            cosp = dotp * _rsqrt(jnp.maximum(cc * xxp, 1e-16))
            cosn = dotn * _rsqrt(jnp.maximum(cc * xxn, 1e-16))
            return acc + ((1.0 - cosp) + jnp.maximum(cosn, 0.0))

        return lax.fori_loop(0, NCTX // 2, jbody, acc)

    fire(0, ci0, ni0, ci2_0, ni2_0, cg0, cr0, nr0, sem_c0, sem_x0, sem_n0)

    def pairbody(t, acc):
        g0 = 2 * t
        g1 = g0 + 1
        fire(g1, ci1, ni1, ci2_1, ni2_1, cg1, cr1, nr1,
             sem_c1, sem_x1, sem_n1)
        drain(g0, ci2_0, ni2_0, cg0, cr0, nr0, sem_c0, sem_x0, sem_n0)
        acc = compute(g0, ci0, ni0, cg0, cr0, nr0, acc)

        @pl.when(t < CHUNKS // 2 - 1)
        def _():
            fire(g1 + 1, ci0, ni0, ci2_0, ni2_0, cg0, cr0, nr0,
                 sem_c0, sem_x0, sem_n0)
        drain(g1, ci2_1, ni2_1, cg1, cr1, nr1, sem_c1, sem_x1, sem_n1)
        acc = compute(g1, ci1, ni1, cg1, cr1, nr1, acc)
        return acc

    acc = lax.fori_loop(0, CHUNKS // 2, pairbody, z)

    outv[...] = acc
    pltpu.sync_copy(outv, out.at[wid])


@jax.jit
def _sc_partials(pc, px, cidx, ctxidx, negidx):
    mesh = plsc.VectorSubcoreMesh(core_axis_name="c", subcore_axis_name="s")
    f = functools.partial(
        pl.kernel,
        out_type=jax.ShapeDtypeStruct((NW, L), jnp.float32),
        mesh=mesh,
        compiler_params=_CP,
        scratch_types=[
            pltpu.VMEM((B_PER_W,), jnp.int32),
            pltpu.VMEM((B_PER_W,), jnp.int32),
            pltpu.VMEM((D, L), jnp.float32),
            pltpu.VMEM((L,), jnp.float32),
            pltpu.VMEM((CH_ROWS,), jnp.int32),
            pltpu.VMEM((CH_ROWS,), jnp.int32),
            pltpu.VMEM((CH_ROWS,), jnp.int32),
            pltpu.VMEM((CH_ROWS,), jnp.int32),
            pltpu.VMEM((CH_ROWS,), jnp.int32),
            pltpu.VMEM((CH_ROWS,), jnp.int32),
            pltpu.VMEM((CH_ROWS,), jnp.int32),
            pltpu.VMEM((CH_ROWS,), jnp.int32),
            pltpu.VMEM((CH_B, 128), jnp.float32),
            pltpu.VMEM((CH_B, 128), jnp.float32),
            pltpu.VMEM((CH_ROWS, 128), jnp.float32),
            pltpu.VMEM((CH_ROWS, 128), jnp.float32),
            pltpu.VMEM((CH_ROWS, 128), jnp.float32),
            pltpu.VMEM((CH_ROWS, 128), jnp.float32),
            pltpu.SemaphoreType.DMA,
            pltpu.SemaphoreType.DMA,
            pltpu.SemaphoreType.DMA,
            pltpu.SemaphoreType.DMA,
            pltpu.SemaphoreType.DMA,
            pltpu.SemaphoreType.DMA,
        ],
    )(_loss_body)
    return f(pc, px, cidx, ctxidx, negidx)


def _tc_reduce_body(x_ref, o_ref):
    s = jnp.sum(x_ref[...]) * (1.0 / (B * NCTX))
    o_ref[...] = jnp.reshape(s, (1, 1))


@jax.jit
def _tc_reduce(partials):
    return pl.pallas_call(
        _tc_reduce_body,
        out_shape=jax.ShapeDtypeStruct((1, 1), jnp.float32),
    )(partials)


def kernel(centers, contexts, center_idxs, context_idxs, neg_idxs):
    cidx = center_idxs.astype(jnp.int32)
    ctxidx = context_idxs.astype(jnp.int32).reshape(-1)
    negidx = neg_idxs.astype(jnp.int32)
    tailc = centers[TAIL_V0:, :].reshape(VOCAB // 2 - TAIL_V0 // 2, 128)
    tailx = contexts[TAIL_V0:, :].reshape(VOCAB // 2 - TAIL_V0 // 2, 128)
    pc, px = _sc_pack(centers.T, contexts.T, tailc, tailx)
    partials = _sc_partials(pc, px, cidx, ctxidx, negidx)
    return _tc_reduce(partials)[0, 0]


# speedup vs baseline: 1.3031x; 1.3031x over previous
import functools

import jax
import jax.numpy as jnp
from jax import lax
from jax.experimental import pallas as pl
from jax.experimental.pallas import tpu as pltpu
from jax.experimental.pallas import tpu_sc as plsc

NC = 2
NS = 16
L = 16
NW = NC * NS

VOCAB = 1000000
D = 64
B = 16384
NCTX = 20

B_PER_W = B // NW
GROUP_B = L
GROUPS = B_PER_W // GROUP_B
GROUP_ROWS = GROUP_B * NCTX
CHUNKS = ((0, 128), (128, 128), (256, 64))


def _rsqrt(s):
    i = plsc.bitcast(s, jnp.int32)
    i = 0x5F3759DF - lax.shift_right_arithmetic(i, 1)
    y = plsc.bitcast(i, jnp.float32)
    h = s * 0.5
    for _ in range(3):
        y = y * (1.5 - h * y * y)
    return y


def _sc_body(centers, contexts, cidx, ctxidx, negidx, out,
             cidx_v, crows, ct, outv,
             ci0, ci1, ni0, ni1, cr0, cr1, nr0, nr1,
             sem_c, sem_x0, sem_x1, sem_n0, sem_n1):
    wid = lax.axis_index("s") * NC + lax.axis_index("c")
    w_base = wid * B_PER_W

    iota = lax.iota(jnp.int32, L)
    iota_nctx = iota * NCTX

    def fire(g, ci, ni, cr, nr, sem_x, sem_n):
        off = (w_base + g * GROUP_B) * NCTX
        pltpu.sync_copy(ctxidx.at[pl.ds(off, GROUP_ROWS)], ci)
        pltpu.sync_copy(negidx.at[pl.ds(off, GROUP_ROWS)], ni)
        for s, n in CHUNKS:
            pltpu.async_copy(contexts.at[ci.at[pl.ds(s, n)]],
                             cr.at[pl.ds(s, n)], sem_x)
            pltpu.async_copy(contexts.at[ni.at[pl.ds(s, n)]],
                             nr.at[pl.ds(s, n)], sem_n)

    def drain(ci, ni, cr, nr, sem_x, sem_n):
        for s, n in CHUNKS:
            pltpu.make_async_copy(contexts.at[ci.at[pl.ds(s, n)]],
                                  cr.at[pl.ds(s, n)], sem_x).wait()
            pltpu.make_async_copy(contexts.at[ni.at[pl.ds(s, n)]],
                                  nr.at[pl.ds(s, n)], sem_n).wait()

    pltpu.sync_copy(cidx.at[pl.ds(w_base, B_PER_W)], cidx_v)
    for k in range(B_PER_W // 128):
        pltpu.async_copy(centers.at[cidx_v.at[pl.ds(k * 128, 128)]],
                         crows.at[pl.ds(k * 128, 128)], sem_c)
    fire(0, ci0, ni0, cr0, nr0, sem_x0, sem_n0)
    for k in range(B_PER_W // 128):
        pltpu.make_async_copy(centers.at[cidx_v.at[pl.ds(k * 128, 128)]],
                              crows.at[pl.ds(k * 128, 128)], sem_c).wait()

    dperm = [jnp.bitwise_and(iota + d, D - 1) for d in range(D)]

    def dperm_dyn(dd):
        return jnp.bitwise_and(iota + dd, D - 1)

    def compute(g, cr, nr, acc):
        base = g * GROUP_B
        rowc = iota + base

        def tbody(t, carry):
            cc0, cc1 = carry
            d0 = t * 8
            for u in range(8):
                cv = plsc.load_gather(crows, [rowc, dperm_dyn(d0 + u)])
                ct[d0 + u] = cv
                if u % 2 == 0:
                    cc0 = cc0 + cv * cv
                else:
                    cc1 = cc1 + cv * cv
            return (cc0, cc1)
        z = jnp.zeros((L,), jnp.float32)
        cc0, cc1 = lax.fori_loop(0, D // 8, tbody, (z, z))
        cc = cc0 + cc1

        def jbody(j, acc):
            rowj = iota_nctx + j
            dp = [z, z]
            xxpa = [z, z]
            dn = [z, z]
            xxna = [z, z]
            for d in range(D):
                k = d & 1
                cv = ct[d]
                xp = plsc.load_gather(cr, [rowj, dperm[d]])
                xn = plsc.load_gather(nr, [rowj, dperm[d]])
                dp[k] = dp[k] + cv * xp
                xxpa[k] = xxpa[k] + xp * xp
                dn[k] = dn[k] + cv * xn
                xxna[k] = xxna[k] + xn * xn
            dotp = dp[0] + dp[1]
            xxp = xxpa[0] + xxpa[1]
            dotn = dn[0] + dn[1]
            xxn = xxna[0] + xxna[1]
            cosp = dotp * _rsqrt(jnp.maximum(cc * xxp, 1e-16))
            cosn = dotn * _rsqrt(jnp.maximum(cc * xxn, 1e-16))
            return acc + ((1.0 - cosp) + jnp.maximum(cosn, 0.0))

        return lax.fori_loop(0, NCTX, jbody, acc)

    def pairbody(t, acc):
        g0 = 2 * t
        g1 = g0 + 1
        fire(g1, ci1, ni1, cr1, nr1, sem_x1, sem_n1)
        drain(ci0, ni0, cr0, nr0, sem_x0, sem_n0)
        acc = compute(g0, cr0, nr0, acc)

        @pl.when(t < GROUPS // 2 - 1)
        def _():
            fire(g1 + 1, ci0, ni0, cr0, nr0, sem_x0, sem_n0)
        drain(ci1, ni1, cr1, nr1, sem_x1, sem_n1)
        acc = compute(g1, cr1, nr1, acc)
        return acc

    acc = lax.fori_loop(0, GROUPS // 2, pairbody,
                        jnp.zeros((L,), jnp.float32))

    outv[...] = acc
    pltpu.sync_copy(outv, out.at[wid])


@jax.jit
def _sc_partials(centers, contexts, cidx, ctxidx, negidx):
    mesh = plsc.VectorSubcoreMesh(core_axis_name="c", subcore_axis_name="s")
    f = functools.partial(
        pl.kernel,
        out_type=jax.ShapeDtypeStruct((NW, L), jnp.float32),
        mesh=mesh,
        compiler_params=pltpu.CompilerParams(needs_layout_passes=False,
                                             use_tc_tiling_on_sc=False),
        scratch_types=[
            pltpu.VMEM((B_PER_W,), jnp.int32),
            pltpu.VMEM((B_PER_W, D), jnp.float32),
            pltpu.VMEM((D, L), jnp.float32),
            pltpu.VMEM((L,), jnp.float32),
            pltpu.VMEM((GROUP_ROWS,), jnp.int32),
            pltpu.VMEM((GROUP_ROWS,), jnp.int32),
            pltpu.VMEM((GROUP_ROWS,), jnp.int32),
            pltpu.VMEM((GROUP_ROWS,), jnp.int32),
            pltpu.VMEM((GROUP_ROWS, D), jnp.float32),
            pltpu.VMEM((GROUP_ROWS, D), jnp.float32),
            pltpu.VMEM((GROUP_ROWS, D), jnp.float32),
            pltpu.VMEM((GROUP_ROWS, D), jnp.float32),
            pltpu.SemaphoreType.DMA,
            pltpu.SemaphoreType.DMA,
            pltpu.SemaphoreType.DMA,
            pltpu.SemaphoreType.DMA,
            pltpu.SemaphoreType.DMA,
        ],
    )(_sc_body)
    return f(centers, contexts, cidx, ctxidx, negidx)


def _tc_reduce_body(x_ref, o_ref):
    s = jnp.sum(x_ref[...]) * (1.0 / (B * NCTX))
    o_ref[...] = jnp.reshape(s, (1, 1))


@jax.jit
def _tc_reduce(partials):
    return pl.pallas_call(
        _tc_reduce_body,
        out_shape=jax.ShapeDtypeStruct((1, 1), jnp.float32),
    )(partials)


def kernel(centers, contexts, center_idxs, context_idxs, neg_idxs):
    cidx = center_idxs.astype(jnp.int32)
    ctxidx = context_idxs.astype(jnp.int32).reshape(-1)
    negidx = neg_idxs.astype(jnp.int32)
    partials = _sc_partials(centers, contexts, cidx, ctxidx, negidx)
    return _tc_reduce(partials)[0, 0]
